# baseline (speedup 1.0000x reference)
import functools

import jax
import jax.numpy as jnp
import numpy as np
from jax import lax
from jax.experimental import pallas as pl
from jax.experimental.pallas import tpu as pltpu
from jax.experimental.pallas import tpu_sc as plsc

_FIELD_DIMS = [100000] * 26
F = 26
K = 16
B = 16384
D_IN = F * K
N = B * F
NW = 32
NPW = N // NW
CH = 1664
NCH = NPW // CH

_OFFSETS = np.concatenate([[0], np.cumsum(_FIELD_DIMS[:-1])]).astype(np.int32)

_mesh = plsc.VectorSubcoreMesh(core_axis_name="c", subcore_axis_name="s")


@functools.partial(
    pl.kernel,
    out_type=(
        jax.ShapeDtypeStruct((NW, NPW, K), jnp.float32),
        jax.ShapeDtypeStruct((NW, NPW), jnp.float32),
    ),
    mesh=_mesh,
    scratch_types=[
        pltpu.VMEM((NPW,), jnp.int32),
        pltpu.VMEM((2, CH, K), jnp.float32),
        pltpu.VMEM((NPW,), jnp.float32),
        pltpu.SemaphoreType.DMA,
        pltpu.SemaphoreType.DMA,
        pltpu.SemaphoreType.DMA,
    ],
    compiler_params=pltpu.CompilerParams(use_tc_tiling_on_sc=False),
)
def _sc_gather(idx_hbm, emb_hbm, fc_hbm, emb_out, fc_out,
               idx_v, ebuf, fc_v, sem_a, sem_b, sem_fc):
    wid = lax.axis_index("s") * 2 + lax.axis_index("c")
    sems = (sem_a, sem_b)

    pltpu.sync_copy(idx_hbm.at[wid], idx_v)
    fc_cp = pltpu.async_copy(fc_hbm.at[idx_v], fc_v, sem_fc)

    copies = [None, None]
    copies[0] = pltpu.async_copy(
        emb_hbm.at[idx_v.at[pl.ds(0, CH)]], ebuf.at[0], sems[0])
    for j in range(NCH):
        if j + 1 < NCH:
            copies[(j + 1) % 2] = pltpu.async_copy(
                emb_hbm.at[idx_v.at[pl.ds((j + 1) * CH, CH)]],
                ebuf.at[(j + 1) % 2], sems[(j + 1) % 2])
        copies[j % 2].wait()
        pltpu.sync_copy(ebuf.at[j % 2], emb_out.at[wid, pl.ds(j * CH, CH)])

    fc_cp.wait()
    pltpu.sync_copy(fc_v, fc_out.at[wid])


BT = 1024


def _tc_body(emb_ref, fc_ref, s_ref, w1_ref, b1_ref, w2_ref, b2_ref,
             w3_ref, bias_ref, out_ref):
    e = emb_ref[...]
    t = jnp.dot(e, s_ref[...], preferred_element_type=jnp.float32)
    fm = 0.5 * (jnp.sum(t * t, axis=1) - jnp.sum(e * e, axis=1))
    fcs = jnp.sum(fc_ref[...], axis=1)
    h = jnp.dot(e, w1_ref[...], preferred_element_type=jnp.float32)
    h = jnp.maximum(h + b1_ref[...], 0.0)
    h = jnp.dot(h, w2_ref[...], preferred_element_type=jnp.float32)
    h = jnp.maximum(h + b2_ref[...], 0.0)
    y = jnp.sum(h * w3_ref[...], axis=1)
    out_ref[...] = fm + fcs + y + bias_ref[0]


def kernel(x, emb_table, fc_table, bias, W1, b1, W2, b2, W3, b3):
    idx = (x + _OFFSETS[None, :]).reshape(NW, NPW)

    emb_rows, fc_vals = _sc_gather(idx, emb_table, fc_table.reshape(-1))
    emb_flat = emb_rows.reshape(B, D_IN)
    fc_mat = fc_vals.reshape(B, F)

    s_pad = jnp.asarray(
        np.equal(np.arange(D_IN)[:, None] % K, np.arange(128)[None, :])
        .astype(np.float32))

    scores = pl.pallas_call(
        _tc_body,
        grid=(B // BT,),
        in_specs=[
            pl.BlockSpec((BT, D_IN), lambda i: (i, 0)),
            pl.BlockSpec((BT, F), lambda i: (i, 0)),
            pl.BlockSpec((D_IN, 128), lambda i: (0, 0)),
            pl.BlockSpec((D_IN, 256), lambda i: (0, 0)),
            pl.BlockSpec((1, 256), lambda i: (0, 0)),
            pl.BlockSpec((256, 128), lambda i: (0, 0)),
            pl.BlockSpec((1, 128), lambda i: (0, 0)),
            pl.BlockSpec((1, 128), lambda i: (0, 0)),
            pl.BlockSpec((1,), lambda i: (0,)),
        ],
        out_specs=pl.BlockSpec((BT,), lambda i: (i,)),
        out_shape=jax.ShapeDtypeStruct((B,), jnp.float32),
    )(emb_flat, fc_mat, s_pad, W1, b1.reshape(1, 256), W2,
      b2.reshape(1, 128), W3.reshape(1, 128), bias)
    return scores
